# baseline (speedup 1.0000x reference)
import functools

import jax
import jax.numpy as jnp
from jax import lax
from jax.experimental import pallas as pl
from jax.experimental.pallas import tpu as pltpu
from jax.experimental.pallas import tpu_sc as plsc

VOCAB_SIZE = 1000000
HIDDEN = 32
BATCH = 4096
HIST = 200

NUM_CORES = 2
NUM_SUBCORES = 16
NW = NUM_CORES * NUM_SUBCORES
NBG = BATCH // 128
NHG = HIDDEN // 8


NCOL_FULL = 7812
COLS_PER_W = NCOL_FULL // NW
NBUF_F = 4


def _fmt_body(w3_hbm, wtail_hbm, w4_hbm, sbuf, rowbuf, tbuf, ssems, osems):
    w = lax.axis_index("s") * NUM_CORES + lax.axis_index("c")
    base = w * COLS_PER_W

    lane = lax.iota(jnp.int32, 16)
    hm_idx = lane & 7
    hg_sel = [lane >> 3, (lane >> 3) + 2]

    def start_stage(c, b):
        for hg in range(NHG):
            pltpu.async_copy(
                w3_hbm.at[hg, :, pl.ds(c * 128, 128)],
                sbuf.at[b, hg, :, pl.ds(0, 128)],
                ssems[b],
            )

    def wait_stage(c, b):
        for hg in range(NHG):
            pltpu.make_async_copy(
                w3_hbm.at[hg, :, pl.ds(c * 128, 128)],
                sbuf.at[b, hg, :, pl.ds(0, 128)],
                ssems[b],
            ).wait()

    def transpose_col(b):
        @plsc.parallel_loop(0, 128, 1, unroll=16)
        def _vm(vm):
            vm_vec = jnp.full((16,), 0, jnp.int32) + vm
            for hh in range(2):
                vals = plsc.load_gather(sbuf.at[b], [hg_sel[hh], hm_idx, vm_vec])
                rowbuf[b, vm >> 2, pl.ds((vm & 3) * 32 + hh * 16, 16)] = vals

    def out_dmas(b, c, start):
        src = rowbuf.at[b]
        dst = w4_hbm.at[pl.ds(32 * c, 32)]
        if start:
            pltpu.async_copy(src, dst, osems[b])
        else:
            pltpu.make_async_copy(src, dst, osems[b]).wait()

    for b in range(NBUF_F):
        start_stage(base + b, b)

    @pl.loop(0, COLS_PER_W, step=NBUF_F)
    def _c2(i0):
        for b in range(NBUF_F):
            c = base + i0 + b
            wait_stage(c, b)

            @pl.when(i0 + b >= NBUF_F)
            def _():
                out_dmas(b, c - NBUF_F, start=False)

            transpose_col(b)
            out_dmas(b, c, start=True)

            @pl.when(i0 + b + NBUF_F < COLS_PER_W)
            def _():
                start_stage(c + NBUF_F, b)

    for b in range(NBUF_F):
        out_dmas(b, base + COLS_PER_W - NBUF_F + b, start=False)

    @pl.when(w < 4)
    def _extra():
        c = NW * COLS_PER_W + w
        start_stage(c, 0)
        wait_stage(c, 0)
        transpose_col(0)
        out_dmas(0, c, start=True)
        out_dmas(0, c, start=False)

    @pl.when(w == 4)
    def _tail():
        pltpu.sync_copy(wtail_hbm, tbuf)
        pltpu.sync_copy(tbuf, w4_hbm.at[pl.ds(32 * NCOL_FULL, 16)])


def _gather_body(idx_hbm, table_hbm, out_hbm, idx_v, rbuf, obuf, gsems, osems):
    w = lax.axis_index("s") * NUM_CORES + lax.axis_index("c")
    pltpu.sync_copy(idx_hbm.at[:, w], idx_v)

    lane = lax.iota(jnp.int32, 16)
    hm_idx = lane & 7
    hg_sel = [lane >> 3, (lane >> 3) + 2]

    def start_gather(t, b):
        pltpu.async_copy(table_hbm.at[idx_v.at[t]], rbuf.at[b], gsems[b])

    start_gather(0, 0)
    start_gather(1, 1)

    def out_slice(t):
        return out_hbm.at[t, :, w]

    def obuf_slice(b):
        return obuf.at[b, :, :, pl.ds(0, 128)]

    @pl.loop(0, HIST, step=2)
    def _t2(t0):
        for b in range(2):
            t = t0 + b
            pltpu.make_async_copy(table_hbm.at[idx_v.at[t]], rbuf.at[b], gsems[b]).wait()
            @pl.when(t >= 2)
            def _():
                pltpu.make_async_copy(obuf_slice(b), out_slice(t), osems[b]).wait()

            @plsc.parallel_loop(0, 128, 1, unroll=8)
            def _bm(bm):
                bm_vec = jnp.full((16,), 0, jnp.int32) + bm
                for hh in range(2):
                    vals = rbuf[b, bm, pl.ds(hh * 16, 16)]
                    plsc.store_scatter(
                        obuf.at[b], [hg_sel[hh], hm_idx, bm_vec], vals
                    )

            pltpu.async_copy(obuf_slice(b), out_slice(t), osems[b])

            @pl.when(t + 2 < HIST)
            def _():
                start_gather(t + 2, b)

    for b in range(2):
        pltpu.make_async_copy(obuf_slice(b), out_slice(HIST - 2 + b), osems[b]).wait()


def _format_call(w3, wtail):
    mesh = plsc.VectorSubcoreMesh(core_axis_name="c", subcore_axis_name="s")
    k = functools.partial(
        pl.kernel,
        out_type=jax.ShapeDtypeStruct((250000, 128), jnp.float32),
        mesh=mesh,
        scratch_types=[
            pltpu.VMEM((NBUF_F, NHG, 8, 129), jnp.float32),
            pltpu.VMEM((NBUF_F, 32, 128), jnp.float32),
            pltpu.VMEM((16, 128), jnp.float32),
            [pltpu.SemaphoreType.DMA] * NBUF_F,
            [pltpu.SemaphoreType.DMA] * NBUF_F,
        ],
        compiler_params=pltpu.CompilerParams(
            use_tc_tiling_on_sc=True, needs_layout_passes=False
        ),
    )(_fmt_body)
    return k(w3, wtail)


def _gather_call(idx3, table):
    mesh = plsc.VectorSubcoreMesh(core_axis_name="c", subcore_axis_name="s")
    k = functools.partial(
        pl.kernel,
        out_type=jax.ShapeDtypeStruct((HIST, NHG, NBG, 8, 128), jnp.float32),
        mesh=mesh,
        scratch_types=[
            pltpu.VMEM((HIST, 128), jnp.int32),
            pltpu.VMEM((2, 128, HIDDEN), jnp.float32),
            pltpu.VMEM((2, NHG, 8, 129), jnp.float32),
            [pltpu.SemaphoreType.DMA] * 2,
            [pltpu.SemaphoreType.DMA] * 2,
        ],
        compiler_params=pltpu.CompilerParams(
            use_tc_tiling_on_sc=False, needs_layout_passes=False
        ),
    )(_gather_body)
    return k(idx3, table)


@jax.jit
def _run(x, weight):
    idx3 = jnp.transpose(x).reshape(HIST, NBG, 128).astype(jnp.int32)
    w3 = jnp.transpose(weight).reshape(NHG, 8, VOCAB_SIZE)
    wtail = weight[NCOL_FULL * 128:].reshape(16, 128)
    w4 = _format_call(w3, wtail)
    w5 = w4.reshape(VOCAB_SIZE, HIDDEN)
    out5 = _gather_call(idx3, w5)
    return out5.transpose(2, 4, 0, 1, 3).reshape(BATCH, HIST, HIDDEN)


def kernel(x, weight):
    return _run(x, weight)


# speedup vs baseline: 1.0057x; 1.0057x over previous
import functools

import jax
import jax.numpy as jnp
from jax import lax
from jax.experimental import pallas as pl
from jax.experimental.pallas import tpu as pltpu
from jax.experimental.pallas import tpu_sc as plsc

VOCAB_SIZE = 1000000
HIDDEN = 32
BATCH = 4096
HIST = 200

NUM_CORES = 2
NUM_SUBCORES = 16
NW = NUM_CORES * NUM_SUBCORES
NBG = BATCH // 128
NHG = HIDDEN // 8


NCOL_FULL = 7812
COLS_PER_W = NCOL_FULL // NW
NBUF_F = 4


def _fmt_body(w3_hbm, wtail_hbm, w4_hbm, sbuf, rowbuf, tbuf, ssems, osems):
    w = lax.axis_index("s") * NUM_CORES + lax.axis_index("c")
    base = w * COLS_PER_W

    lane = lax.iota(jnp.int32, 16)
    hm_idx = lane & 7
    hg_sel = [lane >> 3, (lane >> 3) + 2]

    def start_stage(c, b):
        for hg in range(NHG):
            pltpu.async_copy(
                w3_hbm.at[hg, :, pl.ds(c * 128, 128)],
                sbuf.at[b, hg, :, pl.ds(0, 128)],
                ssems[b],
            )

    def wait_stage(c, b):
        for hg in range(NHG):
            pltpu.make_async_copy(
                w3_hbm.at[hg, :, pl.ds(c * 128, 128)],
                sbuf.at[b, hg, :, pl.ds(0, 128)],
                ssems[b],
            ).wait()

    def transpose_col(b):
        @plsc.parallel_loop(0, 128, 1, unroll=8)
        def _vm(vm):
            vm_vec = jnp.full((16,), 0, jnp.int32) + vm
            for hh in range(2):
                vals = plsc.load_gather(sbuf.at[b], [hg_sel[hh], hm_idx, vm_vec])
                rowbuf[b, vm >> 2, pl.ds((vm & 3) * 32 + hh * 16, 16)] = vals

    def out_dmas(b, c, start):
        src = rowbuf.at[b]
        dst = w4_hbm.at[pl.ds(32 * c, 32)]
        if start:
            pltpu.async_copy(src, dst, osems[b])
        else:
            pltpu.make_async_copy(src, dst, osems[b]).wait()

    for b in range(NBUF_F):
        start_stage(base + b, b)

    @pl.loop(0, COLS_PER_W, step=NBUF_F)
    def _c2(i0):
        for b in range(NBUF_F):
            c = base + i0 + b
            wait_stage(c, b)

            @pl.when(i0 + b >= NBUF_F)
            def _():
                out_dmas(b, c - NBUF_F, start=False)

            transpose_col(b)
            out_dmas(b, c, start=True)

            @pl.when(i0 + b + NBUF_F < COLS_PER_W)
            def _():
                start_stage(c + NBUF_F, b)

    for b in range(NBUF_F):
        out_dmas(b, base + COLS_PER_W - NBUF_F + b, start=False)

    @pl.when(w < 4)
    def _extra():
        c = NW * COLS_PER_W + w
        start_stage(c, 0)
        wait_stage(c, 0)
        transpose_col(0)
        out_dmas(0, c, start=True)
        out_dmas(0, c, start=False)

    @pl.when(w == 4)
    def _tail():
        pltpu.sync_copy(wtail_hbm, tbuf)
        pltpu.sync_copy(tbuf, w4_hbm.at[pl.ds(32 * NCOL_FULL, 16)])


def _gather_body(idx_hbm, table_hbm, out_hbm, idx_v, rbuf, obuf, gsems, osems):
    w = lax.axis_index("s") * NUM_CORES + lax.axis_index("c")
    pltpu.sync_copy(idx_hbm.at[:, w], idx_v)

    lane = lax.iota(jnp.int32, 16)
    hm_idx = lane & 7
    hg_sel = [lane >> 3, (lane >> 3) + 2]

    def start_gather(t, b):
        pltpu.async_copy(table_hbm.at[idx_v.at[t]], rbuf.at[b], gsems[b])

    start_gather(0, 0)
    start_gather(1, 1)

    def out_slice(t):
        return out_hbm.at[t, :, w]

    def obuf_slice(b):
        return obuf.at[b, :, :, pl.ds(0, 128)]

    @pl.loop(0, HIST, step=2)
    def _t2(t0):
        for b in range(2):
            t = t0 + b
            pltpu.make_async_copy(table_hbm.at[idx_v.at[t]], rbuf.at[b], gsems[b]).wait()
            @pl.when(t >= 2)
            def _():
                pltpu.make_async_copy(obuf_slice(b), out_slice(t), osems[b]).wait()

            @plsc.parallel_loop(0, 128, 1, unroll=8)
            def _bm(bm):
                bm_vec = jnp.full((16,), 0, jnp.int32) + bm
                for hh in range(2):
                    vals = rbuf[b, bm, pl.ds(hh * 16, 16)]
                    plsc.store_scatter(
                        obuf.at[b], [hg_sel[hh], hm_idx, bm_vec], vals
                    )

            pltpu.async_copy(obuf_slice(b), out_slice(t), osems[b])

            @pl.when(t + 2 < HIST)
            def _():
                start_gather(t + 2, b)

    for b in range(2):
        pltpu.make_async_copy(obuf_slice(b), out_slice(HIST - 2 + b), osems[b]).wait()


def _format_call(w3, wtail):
    mesh = plsc.VectorSubcoreMesh(core_axis_name="c", subcore_axis_name="s")
    k = functools.partial(
        pl.kernel,
        out_type=jax.ShapeDtypeStruct((250000, 128), jnp.float32),
        mesh=mesh,
        scratch_types=[
            pltpu.VMEM((NBUF_F, NHG, 8, 129), jnp.float32),
            pltpu.VMEM((NBUF_F, 32, 128), jnp.float32),
            pltpu.VMEM((16, 128), jnp.float32),
            [pltpu.SemaphoreType.DMA] * NBUF_F,
            [pltpu.SemaphoreType.DMA] * NBUF_F,
        ],
        compiler_params=pltpu.CompilerParams(
            use_tc_tiling_on_sc=True, needs_layout_passes=False
        ),
    )(_fmt_body)
    return k(w3, wtail)


def _gather_call(idx3, table):
    mesh = plsc.VectorSubcoreMesh(core_axis_name="c", subcore_axis_name="s")
    k = functools.partial(
        pl.kernel,
        out_type=jax.ShapeDtypeStruct((HIST, NHG, NBG, 8, 128), jnp.float32),
        mesh=mesh,
        scratch_types=[
            pltpu.VMEM((HIST, 128), jnp.int32),
            pltpu.VMEM((2, 128, HIDDEN), jnp.float32),
            pltpu.VMEM((2, NHG, 8, 129), jnp.float32),
            [pltpu.SemaphoreType.DMA] * 2,
            [pltpu.SemaphoreType.DMA] * 2,
        ],
        compiler_params=pltpu.CompilerParams(
            use_tc_tiling_on_sc=False, needs_layout_passes=False
        ),
    )(_gather_body)
    return k(idx3, table)


@jax.jit
def _run(x, weight):
    idx3 = jnp.transpose(x).reshape(HIST, NBG, 128).astype(jnp.int32)
    w3 = jnp.transpose(weight).reshape(NHG, 8, VOCAB_SIZE)
    wtail = weight[NCOL_FULL * 128:].reshape(16, 128)
    w4 = _format_call(w3, wtail)
    w5 = w4.reshape(VOCAB_SIZE, HIDDEN)
    out5 = _gather_call(idx3, w5)
    return out5.transpose(2, 4, 0, 1, 3).reshape(BATCH, HIST, HIDDEN)


def kernel(x, weight):
    return _run(x, weight)
